# baseline (speedup 1.0000x reference)
import functools

import jax
import jax.numpy as jnp
from jax import lax
from jax.experimental import pallas as pl
from jax.experimental.pallas import tpu as pltpu
from jax.experimental.pallas import tpu_sc as plsc

N_NODES = 10000
N_EDGES = 320000
DIM = 128
NPAD = 10240

NC = 2
NS = 16
NW = NC * NS
CHUNK = 128
NCHUNKS = 80
EDGES_PER_W = NCHUNKS * CHUNK
E_PAD = NW * EDGES_PER_W
ROWS_PER_S = NPAD // NS


def _sc_body(feat_hbm, src_hbm, dw_hbm, zf_hbm, zd_hbm,
             acc_out, deg_out,
             acc, accd, src_t, dw_a, dw_b,
             rows_a, rows_b, ones_v,
             gsem_a, gsem_b, ssem_a, ssem_b, dwsem_a, dwsem_b, dsem):
    c = lax.axis_index("c")
    s = lax.axis_index("s")
    wid = c * NS + s
    row0 = wid * NCHUNKS

    pltpu.sync_copy(src_hbm.at[pl.ds(row0, NCHUNKS)], src_t)
    pltpu.sync_copy(zf_hbm.at[pl.ds(s * ROWS_PER_S, ROWS_PER_S)],
                    acc.at[pl.ds(s * ROWS_PER_S, ROWS_PER_S)])
    pltpu.sync_copy(zd_hbm.at[pl.ds(s * ROWS_PER_S, ROWS_PER_S)],
                    accd.at[pl.ds(s * ROWS_PER_S, ROWS_PER_S)])
    for i in range(CHUNK // 16):
        ones_v[pl.ds(i * 16, 16)] = jnp.full((16,), 1.0, jnp.float32)

    def gstart(k, buf, sem):
        pltpu.make_async_copy(feat_hbm.at[src_t.at[k, 0]], buf, sem).start()

    def gwait(buf, sem):
        pltpu.make_async_copy(feat_hbm.at[src_t.at[0, 0]], buf, sem).wait()

    def dwstart(k, dw, sem):
        kk = jnp.minimum(k, NCHUNKS - 1)
        pltpu.make_async_copy(dw_hbm.at[pl.ds(row0 + kk, 1)], dw,
                              sem).start()

    def dwwait(dw, sem):
        pltpu.make_async_copy(dw_hbm.at[pl.ds(0, 1)], dw, sem).wait()

    def swait(buf, ssem):
        pltpu.make_async_copy(buf, acc.at[dw_a.at[0, pl.ds(0, CHUNK)]],
                              ssem).wait()

    def process(dw, buf, ssem):
        dv = dw.at[0, pl.ds(0, CHUNK)]
        pltpu.async_copy(ones_v, accd.at[dv], dsem, add=True)

        def scale_body(g, cc):
            w16 = lax.bitcast_convert_type(
                dw[0, pl.ds(CHUNK + g * 16, 16)], jnp.float32)
            for e in range(16):
                wval = w16[e]
                row = g * 16 + e
                for j in range(DIM // 16):
                    sl = pl.ds(j * 16, 16)
                    buf[row, sl] = buf[row, sl] * wval
            return cc

        lax.fori_loop(0, CHUNK // 16, scale_body, 0, unroll=True)
        pltpu.async_copy(buf, acc.at[dv], ssem, add=True)

    dwstart(0, dw_a, dwsem_a)
    gstart(0, rows_a, gsem_a)

    def pipe_body(i, cc):
        k = i * 2
        gwait(rows_a, gsem_a)

        @pl.when(i > 0)
        def _():
            swait(rows_b, ssem_b)

        gstart(k + 1, rows_b, gsem_b)
        dwstart(k + 1, dw_b, dwsem_b)
        dwwait(dw_a, dwsem_a)
        process(dw_a, rows_a, ssem_a)
        gwait(rows_b, gsem_b)
        swait(rows_a, ssem_a)
        gstart(jnp.minimum(k + 2, NCHUNKS - 1), rows_a, gsem_a)
        dwstart(k + 2, dw_a, dwsem_a)
        dwwait(dw_b, dwsem_b)
        process(dw_b, rows_b, ssem_b)
        return cc

    lax.fori_loop(0, NCHUNKS // 2, pipe_body, 0)
    gwait(rows_a, gsem_a)
    dwwait(dw_a, dwsem_a)
    swait(rows_b, ssem_b)
    pltpu.make_async_copy(zf_hbm.at[pl.ds(0, NCHUNKS)],
                          rows_a.at[pl.ds(0, NCHUNKS)], dsem).wait()
    plsc.subcore_barrier()

    sl = pl.ds(s * ROWS_PER_S, ROWS_PER_S)
    pltpu.sync_copy(acc.at[sl], acc_out.at[c, sl])
    pltpu.sync_copy(accd.at[sl], deg_out.at[c, sl])


_sc_aggregate = functools.partial(
    pl.kernel,
    out_type=(
        jax.ShapeDtypeStruct((NC, NPAD, DIM), jnp.float32),
        jax.ShapeDtypeStruct((NC, NPAD), jnp.float32),
    ),
    mesh=plsc.VectorSubcoreMesh(core_axis_name="c", subcore_axis_name="s"),
    scratch_types=[
        pltpu.VMEM_SHARED((NPAD, DIM), jnp.float32),
        pltpu.VMEM_SHARED((NPAD,), jnp.float32),
        pltpu.VMEM((NCHUNKS, 1, CHUNK), jnp.int32),
        pltpu.VMEM((1, 2 * CHUNK), jnp.int32),
        pltpu.VMEM((1, 2 * CHUNK), jnp.int32),
        pltpu.VMEM((CHUNK, DIM), jnp.float32),
        pltpu.VMEM((CHUNK, DIM), jnp.float32),
        pltpu.VMEM((CHUNK,), jnp.float32),
        pltpu.SemaphoreType.DMA,
        pltpu.SemaphoreType.DMA,
        pltpu.SemaphoreType.DMA,
        pltpu.SemaphoreType.DMA,
        pltpu.SemaphoreType.DMA,
        pltpu.SemaphoreType.DMA,
        pltpu.SemaphoreType.DMA,
    ],
)(_sc_body)


def _tc_body(feat_ref, acc_ref, deg_ref, ws_ref, wn_ref, b_ref, out_ref):
    f = feat_ref[...]
    sm = acc_ref[0] + acc_ref[1]
    deg = deg_ref[0] + deg_ref[1]
    h_neigh = sm / jnp.maximum(deg, 1.0)[:, None]
    dn = (((1,), (1,)), ((), ()))
    hn = lax.dot_general(h_neigh, wn_ref[...], dn,
                         preferred_element_type=jnp.float32)
    hs = lax.dot_general(f, ws_ref[...], dn,
                         preferred_element_type=jnp.float32)
    out_ref[...] = hs + hn + b_ref[...]


_TC_BLOCK = 512


def _tc_dense(features, accs, degs, w_self, w_neigh, bias2d):
    grid = (pl.cdiv(N_NODES, _TC_BLOCK),)
    return pl.pallas_call(
        _tc_body,
        grid=grid,
        in_specs=[
            pl.BlockSpec((_TC_BLOCK, DIM), lambda i: (i, 0)),
            pl.BlockSpec((NC, _TC_BLOCK, DIM), lambda i: (0, i, 0)),
            pl.BlockSpec((NC, _TC_BLOCK), lambda i: (0, i)),
            pl.BlockSpec((DIM, DIM), lambda i: (0, 0)),
            pl.BlockSpec((DIM, DIM), lambda i: (0, 0)),
            pl.BlockSpec((1, DIM), lambda i: (0, 0)),
        ],
        out_specs=pl.BlockSpec((_TC_BLOCK, DIM), lambda i: (i, 0)),
        out_shape=jax.ShapeDtypeStruct((N_NODES, DIM), jnp.float32),
    )(features, accs, degs, w_self, w_neigh, bias2d)


def kernel(features, edge_index, edge_weight, W_self, W_neigh, bias):
    npad_e = E_PAD - N_EDGES
    src = jnp.concatenate(
        [edge_index[0].astype(jnp.int32),
         jnp.zeros((npad_e,), jnp.int32)]).reshape(NW * NCHUNKS, 1, CHUNK)
    dst2d = jnp.concatenate(
        [edge_index[1].astype(jnp.int32),
         jnp.full((npad_e,), N_NODES, jnp.int32)]).reshape(
             NW * NCHUNKS, CHUNK)
    w2d = jax.lax.bitcast_convert_type(
        jnp.concatenate([edge_weight.astype(jnp.float32),
                         jnp.zeros((npad_e,), jnp.float32)]),
        jnp.int32).reshape(NW * NCHUNKS, CHUNK)
    dw = jnp.concatenate([dst2d, w2d], axis=1)
    zf = jnp.zeros((NPAD, DIM), jnp.float32)
    zd = jnp.zeros((NPAD,), jnp.float32)
    accs, degs = _sc_aggregate(features, src, dw, zf, zd)
    return _tc_dense(features, accs, degs, W_self, W_neigh,
                     bias.reshape(1, DIM))
